# baseline (speedup 1.0000x reference)
import functools

import jax
import jax.numpy as jnp
from jax import lax
from jax.experimental import pallas as pl
from jax.experimental.pallas import tpu as pltpu
from jax.experimental.pallas import tpu_sc as plsc

_D = 1024
_C = 16
_W = 32
_CHUNK = _C * _D


def _make_sc_kernel(R, S):
    rw = R // _W
    n_chunks = rw // _C

    mesh = plsc.VectorSubcoreMesh(core_axis_name="c", subcore_axis_name="s")

    @functools.partial(
        pl.kernel,
        mesh=mesh,
        out_type=jax.ShapeDtypeStruct((R * _D,), jnp.float32),
        scratch_types=[
            pltpu.VMEM((_CHUNK,), jnp.float32),
            pltpu.VMEM((_CHUNK,), jnp.float32),
            pltpu.VMEM((_CHUNK,), jnp.float32),
            pltpu.VMEM((_CHUNK,), jnp.float32),
            pltpu.SemaphoreType.DMA,
            pltpu.SemaphoreType.DMA,
            pltpu.SemaphoreType.DMA,
            pltpu.SemaphoreType.DMA,
        ],
    )
    def k(x_hbm, pe_hbm, out_hbm, xb0, xb1, pb0, pb1, si0, si1, so0, so1):
        cid = lax.axis_index("c")
        sid = lax.axis_index("s")
        wid = sid * 2 + cid
        row0 = wid * rw
        xbufs, pbufs = (xb0, xb1), (pb0, pb1)
        sins, souts = (si0, si1), (so0, so1)

        pending_in, pending_out = {}, {}

        def start_in(t):
            b = t % 2
            r = row0 + t * _C
            base = r * _D
            peb = (r % S) * _D
            c1 = pltpu.async_copy(x_hbm.at[pl.ds(base, _CHUNK)], xbufs[b], sins[b])
            c2 = pltpu.async_copy(pe_hbm.at[pl.ds(peb, _CHUNK)], pbufs[b], sins[b])
            pending_in[t] = (c1, c2)

        start_in(0)
        for t in range(n_chunks):
            b = t % 2
            if t + 1 < n_chunks:
                if t - 1 >= 0:
                    pending_out.pop(t - 1).wait()
                start_in(t + 1)
            for c in pending_in.pop(t):
                c.wait()

            xb, pb = xbufs[b], pbufs[b]

            @plsc.parallel_loop(0, _CHUNK, 16, unroll=8)
            def add_body(i, xb=xb, pb=pb):
                sl = pl.ds(i, 16)
                plsc.addupdate(xb.at[sl], pb[sl])
            base = (row0 + t * _C) * _D
            pending_out[t] = pltpu.async_copy(
                xb, out_hbm.at[pl.ds(base, _CHUNK)], souts[b]
            )
        for t in sorted(pending_out):
            pending_out.pop(t).wait()

    return k


def kernel(x, pe_weight):
    B, S, D = x.shape
    R = B * S
    x_flat = x.reshape(R * D)
    pe_flat = pe_weight[:S].reshape(S * D)
    out = _make_sc_kernel(R, S)(x_flat, pe_flat)
    return out.reshape(B, S, D)


# speedup vs baseline: 4.6762x; 4.6762x over previous
import jax
import jax.numpy as jnp
from jax.experimental import pallas as pl

_S_BLK = 2048


def _body(x_ref, pe_ref, o_ref):
    o_ref[...] = x_ref[...] + pe_ref[...]


def kernel(x, pe_weight):
    B, S, D = x.shape
    pe = pe_weight[:S]
    grid = (S // _S_BLK, B)
    return pl.pallas_call(
        _body,
        grid=grid,
        in_specs=[
            pl.BlockSpec((1, _S_BLK, D), lambda s, b: (b, s, 0)),
            pl.BlockSpec((_S_BLK, D), lambda s, b: (s, 0)),
        ],
        out_specs=pl.BlockSpec((1, _S_BLK, D), lambda s, b: (b, s, 0)),
        out_shape=jax.ShapeDtypeStruct(x.shape, x.dtype),
    )(x, pe)
